# baseline (speedup 1.0000x reference)
import functools

import jax
import jax.numpy as jnp
from jax import lax
from jax.experimental import pallas as pl
from jax.experimental.pallas import tpu as pltpu
from jax.experimental.pallas import tpu_sc as plsc

B = 16384
L = 20
EMB = 128
HID = 512
R = B * 10


_NC, _NS = 2, 16
_NW = _NC * _NS
_Q = B * L
_PER_W = _Q // _NW
_CHUNK = 128
_NCHUNK = _PER_W // _CHUNK


def _qgather_body(table_hbm, idx_hbm, out_hbm, idx_v, rv0, rv1,
                  gs0, gs1, ss0, ss1):
    wid = lax.axis_index("s") * _NC + lax.axis_index("c")
    base = wid * _PER_W
    pltpu.sync_copy(idx_hbm.at[wid], idx_v)

    def gather(i, rv, gs):
        pltpu.async_copy(table_hbm.at[idx_v.at[i]], rv, gs)

    def gather_wait(rv, gs):
        pltpu.make_async_copy(table_hbm.at[idx_v.at[0]], rv, gs).wait()

    def wb(i, rv, ss):
        pltpu.async_copy(rv, out_hbm.at[pl.ds(base + i * _CHUNK, _CHUNK)],
                         ss)

    def wb_wait(rv, ss):
        pltpu.make_async_copy(rv, out_hbm.at[pl.ds(base, _CHUNK)], ss).wait()

    gather(0, rv0, gs0)
    gather(1, rv1, gs1)
    nj = _NCHUNK // 2

    def body(j, carry):
        i0 = 2 * j
        i1 = i0 + 1
        gather_wait(rv0, gs0)
        wb(i0, rv0, ss0)
        gather_wait(rv1, gs1)
        wb(i1, rv1, ss1)

        @pl.when(j + 1 < nj)
        def _refill():
            wb_wait(rv0, ss0)
            gather(i0 + 2, rv0, gs0)
            wb_wait(rv1, ss1)
            gather(i1 + 2, rv1, gs1)

        return carry

    lax.fori_loop(0, nj, body, 0)
    wb_wait(rv0, ss0)
    wb_wait(rv1, ss1)


def _questions_gather(q_emb, question_flat):
    mesh = plsc.VectorSubcoreMesh(core_axis_name="c", subcore_axis_name="s")
    k = pl.kernel(
        _qgather_body,
        out_type=jax.ShapeDtypeStruct((_Q, EMB), jnp.float32),
        mesh=mesh,
        scratch_types=[
            pltpu.VMEM((_NCHUNK, _CHUNK), jnp.int32),
            pltpu.VMEM((_CHUNK, EMB), jnp.float32),
            pltpu.VMEM((_CHUNK, EMB), jnp.float32),
            pltpu.SemaphoreType.DMA,
            pltpu.SemaphoreType.DMA,
            pltpu.SemaphoreType.DMA,
            pltpu.SemaphoreType.DMA,
        ],
    )
    return k(q_emb, question_flat.reshape(_NW, _NCHUNK, _CHUNK))


def _proj_body(ce_ref, se_ref, me_ref, ze_ref, wc_ref, ws_ref, wm_ref,
               wz_ref, posb_ref, t_ref):
    tc = jnp.dot(ce_ref[...], wc_ref[...], preferred_element_type=jnp.float32)
    ts = jnp.dot(se_ref[...], ws_ref[...], preferred_element_type=jnp.float32)
    tm = jnp.dot(me_ref[...], wm_ref[...], preferred_element_type=jnp.float32)
    tz = jnp.dot(ze_ref[...], wz_ref[...], preferred_element_type=jnp.float32)
    zero = jnp.zeros((56, HID), jnp.float32)
    t = jnp.concatenate([tc, ts, tm, tz, posb_ref[...], zero], axis=0)
    t_ref[...] = t.astype(jnp.bfloat16)


def _build_table(color_emb, shape_emb, material_emb, size_emb, W, b):
    ce = jnp.zeros((16, EMB), jnp.float32).at[:9].set(color_emb)
    se = jnp.zeros((16, EMB), jnp.float32).at[:4].set(shape_emb)
    me = jnp.zeros((16, EMB), jnp.float32).at[:3].set(material_emb)
    ze = jnp.zeros((16, EMB), jnp.float32).at[:3].set(size_emb)
    wc = W[:, 3:131].T
    ws = W[:, 131:259].T
    wm = W[:, 259:387].T
    wz = W[:, 387:515].T
    posb = jnp.zeros((8, HID), jnp.float32).at[:3].set(W[:, :3].T).at[3].set(b)
    return pl.pallas_call(
        _proj_body,
        out_shape=jax.ShapeDtypeStruct((128, HID), jnp.bfloat16),
    )(ce, se, me, ze, wc, ws, wm, wz, posb)


_BBLK = 2048
_NB = B // _BBLK


def _ore_body(code_ref, p_ref, t_ref, o_ref, a_ref):
    j = pl.program_id(1)

    @pl.when(j == 0)
    def _pack():
        code = code_ref[...]
        a_ref[:, 0:10] = (code & 255).astype(jnp.float32)
        a_ref[:, 10:20] = ((code >> 8) & 255).astype(jnp.float32)
        a_ref[:, 20:30] = ((code >> 16) & 255).astype(jnp.float32)
        a_ref[:, 30:40] = (code >> 24).astype(jnp.float32)
        a_ref[:, 40:50] = p_ref[0]
        a_ref[:, 50:60] = p_ref[1]
        a_ref[:, 60:70] = p_ref[2]
    r = lax.broadcasted_iota(jnp.int32, (70, 128), 0)
    col = lax.broadcasted_iota(jnp.int32, (70, 128), 1)
    sel = (((r == j) & (col < 16))
           | ((r == 10 + j) & (col >= 16) & (col < 32))
           | ((r == 20 + j) & (col >= 32) & (col < 48))
           | ((r == 30 + j) & (col >= 48) & (col < 64))
           | ((r == 40 + j) & (col == 64))
           | ((r == 50 + j) & (col == 65))
           | ((r == 60 + j) & (col == 66)))
    rj = jnp.where(sel, 1.0, 0.0)
    y = jnp.dot(a_ref[...], rj, preferred_element_type=jnp.float32)
    colv = lax.broadcasted_iota(jnp.int32, (_BBLK, 128), 1)
    cmod = (colv & 15).astype(jnp.float32)
    hot = jnp.where((colv < 64) & (y == cmod), 1.0, 0.0)
    x = jnp.where(colv < 64, hot,
                  jnp.where(colv < 67, y,
                            jnp.where(colv == 67, 1.0, 0.0)))
    o_ref[...] = jnp.dot(x.astype(jnp.bfloat16), t_ref[...],
                         preferred_element_type=jnp.float32)


def _ore_compute(codes, pos_t, table):
    return pl.pallas_call(
        _ore_body,
        grid=(_NB, 10),
        in_specs=[pl.BlockSpec((_BBLK, 10), lambda i, j: (i, 0)),
                  pl.BlockSpec((3, _BBLK, 10), lambda i, j: (0, i, 0)),
                  pl.BlockSpec((128, HID), lambda i, j: (0, 0))],
        out_specs=pl.BlockSpec((_BBLK, HID), lambda i, j: (j * _NB + i, 0)),
        out_shape=jax.ShapeDtypeStruct((10 * B, HID), jnp.float32),
        scratch_shapes=[pltpu.VMEM((_BBLK, 70), jnp.float32)],
    )(codes, pos_t, table)


_MBLK = 2048


def _mask_body(t_ref, mo_ref, mq_ref, mm_ref):
    row = lax.broadcasted_iota(jnp.int32, (30, _MBLK), 0)
    t = t_ref[...]
    mix = jnp.where(row < 10, (t == 1).astype(jnp.float32),
                    (t == 2).astype(jnp.float32))
    mo_ref[...] = (t[:10, :] == 1).astype(jnp.float32)
    mq_ref[...] = (t[10:, :] == 2).astype(jnp.float32)
    mm_ref[...] = mix


def _masks(types_t):
    return pl.pallas_call(
        _mask_body,
        grid=(B // _MBLK,),
        in_specs=[pl.BlockSpec((30, _MBLK), lambda i: (0, i))],
        out_specs=[pl.BlockSpec((10, _MBLK), lambda i: (0, i)),
                   pl.BlockSpec((20, _MBLK), lambda i: (0, i)),
                   pl.BlockSpec((30, _MBLK), lambda i: (0, i))],
        out_shape=[jax.ShapeDtypeStruct((10, B), jnp.float32),
                   jax.ShapeDtypeStruct((20, B), jnp.float32),
                   jax.ShapeDtypeStruct((30, B), jnp.float32)],
    )(types_t)


def kernel(positions, types, object_positions, object_colors, object_shapes,
           object_materials, object_sizes, question, q_emb, color_emb,
           shape_emb, material_emb, size_emb, W, b):
    table = _build_table(color_emb, shape_emb, material_emb, size_emb, W, b)
    pos_t = object_positions.transpose(2, 0, 1)
    codes = (object_colors | (object_shapes << 8) | (object_materials << 16)
             | (object_sizes << 24))
    ore_flat = _ore_compute(codes, pos_t, table)
    ore = ore_flat.reshape(10, B, HID).transpose(1, 0, 2)
    questions_flat = _questions_gather(q_emb, question.T)
    questions = questions_flat.reshape(L, B, EMB).transpose(1, 0, 2)
    mo, mq, mm = _masks(types.T)
    object_mask = mo.T.reshape(B, 1, 1, 10)
    question_mask = mq.T.reshape(B, 1, 1, 20)
    mixed_mask = mm.T.reshape(B, 1, 1, 30)
    return (ore, questions, object_mask, question_mask, mixed_mask)


# speedup vs baseline: 1.0991x; 1.0991x over previous
import functools

import jax
import jax.numpy as jnp
from jax import lax
from jax.experimental import pallas as pl
from jax.experimental.pallas import tpu as pltpu
from jax.experimental.pallas import tpu_sc as plsc

B = 16384
L = 20
EMB = 128
HID = 512
R = B * 10


_NC, _NS = 2, 16
_NW = _NC * _NS
_Q = B * L
_PER_W = _Q // _NW
_CHUNK = 128
_NCHUNK = _PER_W // _CHUNK


def _qgather_body(table_hbm, idx_hbm, out_hbm, idx_v, rv0, rv1,
                  gs0, gs1, ss0, ss1):
    wid = lax.axis_index("s") * _NC + lax.axis_index("c")
    base = wid * _PER_W
    pltpu.sync_copy(idx_hbm.at[wid], idx_v)

    def gather(i, rv, gs):
        pltpu.async_copy(table_hbm.at[idx_v.at[i]], rv, gs)

    def gather_wait(rv, gs):
        pltpu.make_async_copy(table_hbm.at[idx_v.at[0]], rv, gs).wait()

    def wb(i, rv, ss):
        pltpu.async_copy(rv, out_hbm.at[pl.ds(base + i * _CHUNK, _CHUNK)],
                         ss)

    def wb_wait(rv, ss):
        pltpu.make_async_copy(rv, out_hbm.at[pl.ds(base, _CHUNK)], ss).wait()

    gather(0, rv0, gs0)
    gather(1, rv1, gs1)
    nj = _NCHUNK // 2

    def body(j, carry):
        i0 = 2 * j
        i1 = i0 + 1
        gather_wait(rv0, gs0)
        wb(i0, rv0, ss0)
        gather_wait(rv1, gs1)
        wb(i1, rv1, ss1)

        @pl.when(j + 1 < nj)
        def _refill():
            wb_wait(rv0, ss0)
            gather(i0 + 2, rv0, gs0)
            wb_wait(rv1, ss1)
            gather(i1 + 2, rv1, gs1)

        return carry

    lax.fori_loop(0, nj, body, 0)
    wb_wait(rv0, ss0)
    wb_wait(rv1, ss1)


def _questions_gather(q_emb, question_flat):
    mesh = plsc.VectorSubcoreMesh(core_axis_name="c", subcore_axis_name="s")
    k = pl.kernel(
        _qgather_body,
        out_type=jax.ShapeDtypeStruct((_Q, EMB), jnp.float32),
        mesh=mesh,
        scratch_types=[
            pltpu.VMEM((_NCHUNK, _CHUNK), jnp.int32),
            pltpu.VMEM((_CHUNK, EMB), jnp.float32),
            pltpu.VMEM((_CHUNK, EMB), jnp.float32),
            pltpu.SemaphoreType.DMA,
            pltpu.SemaphoreType.DMA,
            pltpu.SemaphoreType.DMA,
            pltpu.SemaphoreType.DMA,
        ],
    )
    return k(q_emb, question_flat.reshape(_NW, _NCHUNK, _CHUNK))


def _proj_body(ce_ref, se_ref, me_ref, ze_ref, wc_ref, ws_ref, wm_ref,
               wz_ref, posb_ref, t_ref):
    tc = jnp.dot(ce_ref[...], wc_ref[...], preferred_element_type=jnp.float32)
    ts = jnp.dot(se_ref[...], ws_ref[...], preferred_element_type=jnp.float32)
    tm = jnp.dot(me_ref[...], wm_ref[...], preferred_element_type=jnp.float32)
    tz = jnp.dot(ze_ref[...], wz_ref[...], preferred_element_type=jnp.float32)
    zero = jnp.zeros((56, HID), jnp.float32)
    t = jnp.concatenate([tc, ts, tm, tz, posb_ref[...], zero], axis=0)
    t_ref[...] = t.astype(jnp.bfloat16)


def _build_table(color_emb, shape_emb, material_emb, size_emb, W, b):
    ce = jnp.zeros((16, EMB), jnp.float32).at[:9].set(color_emb)
    se = jnp.zeros((16, EMB), jnp.float32).at[:4].set(shape_emb)
    me = jnp.zeros((16, EMB), jnp.float32).at[:3].set(material_emb)
    ze = jnp.zeros((16, EMB), jnp.float32).at[:3].set(size_emb)
    wc = W[:, 3:131].T
    ws = W[:, 131:259].T
    wm = W[:, 259:387].T
    wz = W[:, 387:515].T
    posb = jnp.zeros((8, HID), jnp.float32).at[:3].set(W[:, :3].T).at[3].set(b)
    return pl.pallas_call(
        _proj_body,
        out_shape=jax.ShapeDtypeStruct((128, HID), jnp.bfloat16),
    )(ce, se, me, ze, wc, ws, wm, wz, posb)


_BBLK = 4096
_NB = B // _BBLK


def _ore_body(c_ref, s_ref, m_ref, z_ref, p_ref, t_ref, o_ref, a_ref):
    j = pl.program_id(1)

    @pl.when(j == 0)
    def _pack():
        a_ref[:, 0:10] = c_ref[...].astype(jnp.float32)
        a_ref[:, 10:20] = s_ref[...].astype(jnp.float32)
        a_ref[:, 20:30] = m_ref[...].astype(jnp.float32)
        a_ref[:, 30:40] = z_ref[...].astype(jnp.float32)
        a_ref[:, 40:50] = p_ref[0]
        a_ref[:, 50:60] = p_ref[1]
        a_ref[:, 60:70] = p_ref[2]
    r = lax.broadcasted_iota(jnp.int32, (70, 128), 0)
    col = lax.broadcasted_iota(jnp.int32, (70, 128), 1)
    sel = (((r == j) & (col < 16))
           | ((r == 10 + j) & (col >= 16) & (col < 32))
           | ((r == 20 + j) & (col >= 32) & (col < 48))
           | ((r == 30 + j) & (col >= 48) & (col < 64))
           | ((r == 40 + j) & (col == 64))
           | ((r == 50 + j) & (col == 65))
           | ((r == 60 + j) & (col == 66)))
    rj = jnp.where(sel, 1.0, 0.0)
    y = jnp.dot(a_ref[...], rj, preferred_element_type=jnp.float32)
    colv = lax.broadcasted_iota(jnp.int32, (_BBLK, 128), 1)
    cmod = (colv & 15).astype(jnp.float32)
    hot = jnp.where((colv < 64) & (y == cmod), 1.0, 0.0)
    x = jnp.where(colv < 64, hot,
                  jnp.where(colv < 67, y,
                            jnp.where(colv == 67, 1.0, 0.0)))
    o_ref[...] = jnp.dot(x.astype(jnp.bfloat16), t_ref[...],
                         preferred_element_type=jnp.float32)


def _ore_compute(colors, shapes, materials, sizes, pos_t, table):
    bs = pl.BlockSpec((_BBLK, 10), lambda i, j: (i, 0))
    return pl.pallas_call(
        _ore_body,
        grid=(_NB, 10),
        in_specs=[bs, bs, bs, bs,
                  pl.BlockSpec((3, _BBLK, 10), lambda i, j: (0, i, 0)),
                  pl.BlockSpec((128, HID), lambda i, j: (0, 0))],
        out_specs=pl.BlockSpec((_BBLK, HID), lambda i, j: (j * _NB + i, 0)),
        out_shape=jax.ShapeDtypeStruct((10 * B, HID), jnp.float32),
        scratch_shapes=[pltpu.VMEM((_BBLK, 70), jnp.float32)],
    )(colors, shapes, materials, sizes, pos_t, table)


_MBLK = 2048


def _mask_body(t_ref, mo_ref, mq_ref, mm_ref):
    row = lax.broadcasted_iota(jnp.int32, (30, _MBLK), 0)
    t = t_ref[...]
    mix = jnp.where(row < 10, (t == 1).astype(jnp.float32),
                    (t == 2).astype(jnp.float32))
    mo_ref[...] = (t[:10, :] == 1).astype(jnp.float32)
    mq_ref[...] = (t[10:, :] == 2).astype(jnp.float32)
    mm_ref[...] = mix


def _masks(types_t):
    return pl.pallas_call(
        _mask_body,
        grid=(B // _MBLK,),
        in_specs=[pl.BlockSpec((30, _MBLK), lambda i: (0, i))],
        out_specs=[pl.BlockSpec((10, _MBLK), lambda i: (0, i)),
                   pl.BlockSpec((20, _MBLK), lambda i: (0, i)),
                   pl.BlockSpec((30, _MBLK), lambda i: (0, i))],
        out_shape=[jax.ShapeDtypeStruct((10, B), jnp.float32),
                   jax.ShapeDtypeStruct((20, B), jnp.float32),
                   jax.ShapeDtypeStruct((30, B), jnp.float32)],
    )(types_t)


def kernel(positions, types, object_positions, object_colors, object_shapes,
           object_materials, object_sizes, question, q_emb, color_emb,
           shape_emb, material_emb, size_emb, W, b):
    table = _build_table(color_emb, shape_emb, material_emb, size_emb, W, b)
    pos_t = object_positions.transpose(2, 0, 1)
    ore_flat = _ore_compute(object_colors, object_shapes, object_materials,
                            object_sizes, pos_t, table)
    ore = ore_flat.reshape(10, B, HID).transpose(1, 0, 2)
    questions_flat = _questions_gather(q_emb, question.T)
    questions = questions_flat.reshape(L, B, EMB).transpose(1, 0, 2)
    mo, mq, mm = _masks(types.T)
    object_mask = mo.T.reshape(B, 1, 1, 10)
    question_mask = mq.T.reshape(B, 1, 1, 20)
    mixed_mask = mm.T.reshape(B, 1, 1, 30)
    return (ore, questions, object_mask, question_mask, mixed_mask)
